# Initial kernel scaffold; baseline (speedup 1.0000x reference)
import jax
import jax.numpy as jnp
from jax.experimental import pallas as pl


def kernel(predicted_locs, predicted_scores, boxes, labels, priors_cxcycz):
    raise NotImplementedError("write your pallas kernel here")


# speedup vs baseline: 9.2354x; 9.2354x over previous
import jax
import jax.numpy as jnp
from jax import lax
from jax.experimental import pallas as pl
from jax.experimental.pallas import tpu as pltpu

_THRESHOLD = 0.5
_NEG_POS_RATIO = 3
_LANES = 128


def _mbl_body(priors_ref, boxmeta_ref, locs_ref, scores_ref,
              conf_out, loc_out, cbits_ref, npos_ref, acc_ref,
              *, n_batch, n_priors, n_obj, n_cls, rows):
    b = pl.program_id(0)
    f32 = jnp.float32
    i32 = jnp.int32

    pidx = (lax.broadcasted_iota(i32, (rows, _LANES), 0) * _LANES
            + lax.broadcasted_iota(i32, (rows, _LANES), 1))
    valid = pidx < n_priors

    pc = [priors_ref[d] for d in range(3)]
    ps = [priors_ref[3 + d] for d in range(3)]
    plo = [pc[d] - ps[d] / 2.0 for d in range(3)]
    phi = [pc[d] + ps[d] / 2.0 for d in range(3)]
    pvol = ((phi[0] - plo[0]) * (phi[1] - plo[1])) * (phi[2] - plo[2])

    blo = [[boxmeta_ref[0, d, o] for d in range(3)] for o in range(n_obj)]
    bhi = [[boxmeta_ref[0, 3 + d, o] for d in range(3)] for o in range(n_obj)]
    blab = [boxmeta_ref[0, 6, o] for o in range(n_obj)]

    mx = jnp.full((rows, _LANES), -1.0, f32)
    arg = jnp.zeros((rows, _LANES), i32)
    pfe = []
    big = jnp.int32(2**30)
    for o in range(n_obj):
        v1 = ((bhi[o][0] - blo[o][0]) * (bhi[o][1] - blo[o][1])
              * (bhi[o][2] - blo[o][2]))
        inter = None
        for d in range(3):
            e = jnp.maximum(jnp.minimum(bhi[o][d], phi[d])
                            - jnp.maximum(blo[o][d], plo[d]), 0.0)
            inter = e if inter is None else inter * e
        ov = inter / (v1 + pvol - inter)
        ov = jnp.where(valid, ov, 0.0)
        gt = ov > mx
        mx = jnp.where(gt, ov, mx)
        arg = jnp.where(gt, o, arg)
        mo = jnp.max(ov)
        pfe.append(jnp.min(jnp.where(ov == mo, pidx, big)))

    for o in range(n_obj):
        m = pidx == pfe[o]
        arg = jnp.where(m, o, arg)
        mx = jnp.where(m, 1.0, mx)

    lbl = jnp.zeros((rows, _LANES), f32)
    bc = [jnp.zeros((rows, _LANES), f32) for _ in range(3)]
    bs = [jnp.zeros((rows, _LANES), f32) for _ in range(3)]
    for o in range(n_obj):
        sel = arg == o
        lbl = jnp.where(sel, blab[o], lbl)
        for d in range(3):
            bc[d] = jnp.where(sel, (blo[o][d] + bhi[o][d]) / 2.0, bc[d])
            bs[d] = jnp.where(sel, bhi[o][d] - blo[o][d], bs[d])
    lbl = jnp.where(mx < _THRESHOLD, 0.0, lbl)
    posf = jnp.where((lbl > 0.0) & valid, 1.0, 0.0)

    adiff = None
    for d in range(3):
        g = (bc[d] - pc[d]) / (ps[d] / 10.0)
        a = jnp.abs(locs_ref[0, d] - g)
        adiff = a if adiff is None else adiff + a
    for d in range(3):
        g = jnp.log(bs[d] / ps[d]) * 5.0
        adiff = adiff + jnp.abs(locs_ref[0, 3 + d] - g)
    loc_b = jnp.sum(adiff * posf)

    xs = [scores_ref[0, c] for c in range(n_cls)]
    m = xs[0]
    for c in range(1, n_cls):
        m = jnp.maximum(m, xs[c])
    s = jnp.exp(xs[0] - m)
    for c in range(1, n_cls):
        s = s + jnp.exp(xs[c] - m)
    logz = m + jnp.log(s)
    ltgt = jnp.zeros((rows, _LANES), f32)
    for c in range(n_cls):
        ltgt = jnp.where(lbl == float(c), xs[c], ltgt)
    conf = logz - ltgt
    conf_pos_b = jnp.sum(conf * posf)
    cneg = jnp.where(valid & (posf == 0.0), conf, 0.0)

    cbits_ref[pl.ds(b, 1)] = lax.bitcast_convert_type(cneg, i32)[None]
    npos_ref[b] = jnp.sum(posf)

    @pl.when(b == 0)
    def _init():
        acc_ref[0] = 0.0
        acc_ref[1] = 0.0

    acc_ref[0] += loc_b
    acc_ref[1] += conf_pos_b

    @pl.when(b == n_batch - 1)
    def _finish():
        hard = jnp.float32(0.0)
        npt = jnp.float32(0.0)
        for bb in range(n_batch):
            vb = cbits_ref[bb]
            k = npos_ref[bb] * float(_NEG_POS_RATIO)
            ki = k.astype(i32)

            def step(_, carry):
                lo, hi = carry
                mid = lo + lax.div(hi - lo + 1, 2)
                cnt = jnp.sum(jnp.where(vb >= mid, 1, 0))
                ge = cnt >= ki
                return (jnp.where(ge, mid, lo),
                        jnp.where(ge, hi, mid - 1))

            lo, _hi = lax.fori_loop(
                0, 31, step, (jnp.int32(0), jnp.int32(0x7f800000)))
            t = lax.bitcast_convert_type(lo, f32)
            vf = lax.bitcast_convert_type(vb, f32)
            gtm = vb > lo
            cnt_gt = jnp.sum(jnp.where(gtm, 1.0, 0.0))
            sum_gt = jnp.sum(jnp.where(gtm, vf, 0.0))
            topk = jnp.where(ki > 0, sum_gt + (k - cnt_gt) * t, 0.0)
            hard = hard + topk
            npt = npt + npos_ref[bb]
        conf_out[...] = jnp.full((1, 1), (hard + acc_ref[1]) / npt, f32)
        loc_out[...] = jnp.full((1, 1), acc_ref[0] / (npt * 6.0), f32)


def kernel(predicted_locs, predicted_scores, boxes, labels, priors_cxcycz):
    B, P, C = predicted_scores.shape
    O = boxes.shape[1]
    rows = (P + _LANES - 1) // _LANES
    rows = ((rows + 7) // 8) * 8
    ppad = rows * _LANES

    locs_t = jnp.pad(jnp.transpose(predicted_locs, (0, 2, 1)),
                     ((0, 0), (0, 0), (0, ppad - P))).reshape(B, 6, rows, _LANES)
    scores_t = jnp.pad(jnp.transpose(predicted_scores, (0, 2, 1)),
                       ((0, 0), (0, 0), (0, ppad - P))).reshape(B, C, rows, _LANES)
    pri_t = jnp.pad(jnp.transpose(priors_cxcycz, (1, 0)),
                    ((0, 0), (0, ppad - P)),
                    constant_values=1.0).reshape(6, rows, _LANES)
    boxmeta = jnp.concatenate(
        [jnp.transpose(boxes, (0, 2, 1)),
         labels.astype(jnp.float32)[:, None, :],
         jnp.zeros((B, 1, O), jnp.float32)], axis=1)

    import functools
    body = functools.partial(_mbl_body, n_batch=B, n_priors=P, n_obj=O,
                             n_cls=C, rows=rows)
    conf, loc = pl.pallas_call(
        body,
        grid=(B,),
        in_specs=[
            pl.BlockSpec((6, rows, _LANES), lambda b: (0, 0, 0)),
            pl.BlockSpec((1, 8, O), lambda b: (b, 0, 0),
                         memory_space=pltpu.MemorySpace.SMEM),
            pl.BlockSpec((1, 6, rows, _LANES), lambda b: (b, 0, 0, 0)),
            pl.BlockSpec((1, C, rows, _LANES), lambda b: (b, 0, 0, 0)),
        ],
        out_specs=[
            pl.BlockSpec((1, 1), lambda b: (0, 0)),
            pl.BlockSpec((1, 1), lambda b: (0, 0)),
        ],
        out_shape=[
            jax.ShapeDtypeStruct((1, 1), jnp.float32),
            jax.ShapeDtypeStruct((1, 1), jnp.float32),
        ],
        scratch_shapes=[
            pltpu.VMEM((B, rows, _LANES), jnp.int32),
            pltpu.SMEM((B,), jnp.float32),
            pltpu.SMEM((2,), jnp.float32),
        ],
        compiler_params=pltpu.CompilerParams(
            dimension_semantics=("arbitrary",)),
    )(pri_t, boxmeta, locs_t, scores_t)
    return (jnp.reshape(conf, ()), jnp.reshape(loc, ()))
